# baseline (speedup 1.0000x reference)
import functools

import numpy as np
import jax
import jax.numpy as jnp
from jax import lax
from jax.experimental import pallas as pl
from jax.experimental.pallas import tpu as pltpu
from jax.experimental.pallas import tpu_sc as plsc

NS = 64
NCAT = 8
NL = 10000
NR = 10000
E = 160000
DIMS = [64, 112, 160]

NSC = 2
NSUB = 16
NWORK = NSC * NSUB
CH = 128
EP = 163840
NCH_W = EP // CH // NWORK
W_ROWS = EP // NWORK
NP = 10240
ACC_ROWS = NP // NSUB
D = 128

_SC_MESH = dict(core_axis_name="c", subcore_axis_name="s",
                num_cores=NSC, num_subcores=NSUB)


def _sc_gather(table, idx2d, gw):
    mesh = plsc.VectorSubcoreMesh(**_SC_MESH)

    @functools.partial(
        pl.kernel,
        out_type=jax.ShapeDtypeStruct((EP, gw), jnp.float32),
        mesh=mesh,
        compiler_params=pltpu.CompilerParams(use_tc_tiling_on_sc=False),
        scratch_types=[
            pltpu.VMEM((NCH_W, CH), jnp.int32),
            pltpu.VMEM((CH, gw), jnp.float32),
            pltpu.VMEM((CH, gw), jnp.float32),
            pltpu.VMEM((CH, gw), jnp.float32),
            pltpu.VMEM((CH, gw), jnp.float32),
            pltpu.SemaphoreType.DMA,
            pltpu.SemaphoreType.DMA,
            pltpu.SemaphoreType.DMA,
            pltpu.SemaphoreType.DMA,
        ],
    )
    def k(table_h, idx_h, out_h, idx_v, b0, b1, b2, b3, s0, s1, s2, s3):
        cid = lax.axis_index("c")
        sid = lax.axis_index("s")
        wid = sid * NSC + cid
        pltpu.sync_copy(idx_h.at[pl.ds(wid * NCH_W, NCH_W)], idx_v)
        base = wid * W_ROWS
        bufs = (b0, b1, b2, b3)
        sems = (s0, s1, s2, s3)
        for t in range(3):
            pltpu.async_copy(table_h.at[idx_v.at[t]], bufs[t], sems[t])

        def body(i, carry):
            for t in range(4):
                j = 4 * i + t
                bn = (t + 3) % 4

                @pl.when(j + 3 < NCH_W)
                def _():
                    pltpu.async_copy(table_h.at[idx_v.at[j + 3]], bufs[bn], sems[bn])

                pltpu.make_async_copy(table_h.at[idx_v.at[j]], bufs[t], sems[t]).wait()
                pltpu.sync_copy(bufs[t], out_h.at[pl.ds(base + j * CH, CH)])
            return carry

        lax.fori_loop(0, NCH_W // 4, body, 0)

    return k(table, idx2d)


def _sc_edge_vec(tab_s, tab_d, idxs2d, idxd2d):
    mesh = plsc.VectorSubcoreMesh(**_SC_MESH)

    @functools.partial(
        pl.kernel,
        out_type=jax.ShapeDtypeStruct((EP, 16), jnp.float32),
        mesh=mesh,
        compiler_params=pltpu.CompilerParams(use_tc_tiling_on_sc=False),
        scratch_types=[
            pltpu.VMEM((NCH_W, CH), jnp.int32),
            pltpu.VMEM((NCH_W, CH), jnp.int32),
            pltpu.VMEM((CH, 16), jnp.float32),
            pltpu.VMEM((CH, 16), jnp.float32),
            pltpu.VMEM((CH, 16), jnp.float32),
            pltpu.VMEM((CH, 16), jnp.float32),
            pltpu.VMEM((CH, 16), jnp.float32),
            pltpu.VMEM((CH, 16), jnp.float32),
            pltpu.SemaphoreType.DMA,
            pltpu.SemaphoreType.DMA,
            pltpu.SemaphoreType.DMA,
            pltpu.SemaphoreType.DMA,
        ],
    )
    def k(tabs_h, tabd_h, idxs_h, idxd_h, out_h, idxs_v, idxd_v,
          bs0, bd0, bs1, bd1, ov0, ov1, ss0, sd0, ss1, sd1):
        cid = lax.axis_index("c")
        sid = lax.axis_index("s")
        wid = sid * NSC + cid
        pltpu.sync_copy(idxs_h.at[pl.ds(wid * NCH_W, NCH_W)], idxs_v)
        pltpu.sync_copy(idxd_h.at[pl.ds(wid * NCH_W, NCH_W)], idxd_v)
        base = wid * W_ROWS
        pltpu.async_copy(tabs_h.at[idxs_v.at[0]], bs0, ss0)
        pltpu.async_copy(tabd_h.at[idxd_v.at[0]], bd0, sd0)

        def diff(bs, bd, ov):
            def row(r, carry):
                for u in range(4):
                    ov[4 * r + u, 0:16] = bd[4 * r + u, 0:16] - bs[4 * r + u, 0:16]
                return carry
            lax.fori_loop(0, CH // 4, row, 0)

        def body(i, carry):
            j0 = 2 * i
            pltpu.async_copy(tabs_h.at[idxs_v.at[j0 + 1]], bs1, ss1)
            pltpu.async_copy(tabd_h.at[idxd_v.at[j0 + 1]], bd1, sd1)
            pltpu.make_async_copy(tabs_h.at[idxs_v.at[j0]], bs0, ss0).wait()
            pltpu.make_async_copy(tabd_h.at[idxd_v.at[j0]], bd0, sd0).wait()
            diff(bs0, bd0, ov0)
            pltpu.sync_copy(ov0, out_h.at[pl.ds(base + j0 * CH, CH)])

            @pl.when(j0 + 2 < NCH_W)
            def _():
                pltpu.async_copy(tabs_h.at[idxs_v.at[j0 + 2]], bs0, ss0)
                pltpu.async_copy(tabd_h.at[idxd_v.at[j0 + 2]], bd0, sd0)

            pltpu.make_async_copy(tabs_h.at[idxs_v.at[j0 + 1]], bs1, ss1).wait()
            pltpu.make_async_copy(tabd_h.at[idxd_v.at[j0 + 1]], bd1, sd1).wait()
            diff(bs1, bd1, ov1)
            pltpu.sync_copy(ov1, out_h.at[pl.ds(base + (j0 + 1) * CH, CH)])
            return carry

        lax.fori_loop(0, NCH_W // 2, body, 0)

    return k(tab_s, tab_d, idxs2d, idxd2d)


def _sc_scatter_add(msg, idx2d, zeros_np):
    mesh = plsc.VectorSubcoreMesh(**_SC_MESH)

    @functools.partial(
        pl.kernel,
        out_type=jax.ShapeDtypeStruct((NSC, NP, D), jnp.float32),
        mesh=mesh,
        scratch_types=[
            pltpu.VMEM((NCH_W, CH), jnp.int32),
            pltpu.VMEM((CH, D), jnp.float32),
            pltpu.VMEM((CH, D), jnp.float32),
            pltpu.VMEM_SHARED((NP, D), jnp.float32),
            pltpu.SemaphoreType.DMA,
            pltpu.SemaphoreType.DMA,
        ],
    )
    def k(msg_h, idx_h, zero_h, out_h, idx_v, b0, b1, acc, s0, s1):
        cid = lax.axis_index("c")
        sid = lax.axis_index("s")
        wid = cid * NSUB + sid
        pltpu.sync_copy(idx_h.at[pl.ds(wid * NCH_W, NCH_W)], idx_v)
        pltpu.sync_copy(zero_h.at[pl.ds(sid * ACC_ROWS, ACC_ROWS)],
                        acc.at[pl.ds(sid * ACC_ROWS, ACC_ROWS)])
        plsc.subcore_barrier()
        base = wid * W_ROWS
        bufs = (b0, b1)
        sems = (s0, s1)
        pltpu.async_copy(msg_h.at[pl.ds(base, CH)], b0, s0)

        def body(i, carry):
            for t in range(2):
                j = 2 * i + t
                bn = (t + 1) % 2

                @pl.when(j + 1 < NCH_W)
                def _():
                    pltpu.async_copy(msg_h.at[pl.ds(base + (j + 1) * CH, CH)],
                                     bufs[bn], sems[bn])

                pltpu.make_async_copy(msg_h.at[pl.ds(base, CH)], bufs[t], sems[t]).wait()
                pltpu.sync_copy(bufs[t], acc.at[idx_v.at[j]], add=True)
            return carry

        lax.fori_loop(0, NCH_W // 2, body, 0)
        plsc.subcore_barrier()
        pltpu.sync_copy(acc.at[pl.ds(sid * ACC_ROWS, ACC_ROWS)],
                        out_h.at[cid, pl.ds(sid * ACC_ROWS, ACC_ROWS)])

    return k(msg, idx2d, zeros_np)


_NB_NODE = 5
_BN = NL // _NB_NODE
_BE = 2048
_NB_EDGE = EP // _BE

_BF = jnp.bfloat16


def _tc_atom_enc(x, emb):
    def body(x_ref, emb_ref, out_ref):
        xv = x_ref[...]
        acc = jnp.zeros((_BN, NS), jnp.float32)
        ids = lax.broadcasted_iota(jnp.int32, (_BN, 16), 1)
        for i in range(NCAT):
            oh = (xv[:, i:i + 1] == ids).astype(_BF)
            acc = acc + jnp.dot(oh, emb_ref[i], preferred_element_type=jnp.float32)
        out_ref[...] = acc

    return pl.pallas_call(
        body,
        grid=(_NB_NODE,),
        in_specs=[pl.BlockSpec((_BN, NCAT), lambda i: (i, 0)),
                  pl.BlockSpec((NCAT, 16, NS), lambda i: (0, 0, 0))],
        out_specs=pl.BlockSpec((_BN, NS), lambda i: (i, 0)),
        out_shape=jax.ShapeDtypeStruct((NL, NS), jnp.float32),
    )(x, emb.astype(_BF))


def _tc_geometry(vec_in, ef, W1, b1, W2, b2, start, stop):
    offs = np.linspace(start, stop, 32).astype(np.float32)
    coeff = float(-0.5 / (offs[1] - offs[0]) ** 2)
    s3 = float(np.sqrt(3.0))

    def body(*refs):
        if ef is not None:
            vec_ref, ef_ref, offs_ref, w1_ref, b1_ref, w2_ref, b2_ref, ea_ref, sh_ref = refs
        else:
            vec_ref, offs_ref, w1_ref, b1_ref, w2_ref, b2_ref, ea_ref, sh_ref = refs
        vec = vec_ref[...]
        x = vec[:, 0:1]
        y = vec[:, 1:2]
        z = vec[:, 2:3]
        d2 = x * x + y * y + z * z
        d = jnp.sqrt(d2 + 1e-12)
        g = jnp.exp(coeff * (d - offs_ref[...]) ** 2)
        if ef is not None:
            g = jnp.concatenate([ef_ref[...], g], axis=-1)
        h = jnp.maximum(jnp.dot(g.astype(_BF), w1_ref[...],
                                preferred_element_type=jnp.float32)
                        + b1_ref[...], 0.0)
        ea = jnp.dot(h.astype(_BF), w2_ref[...],
                     preferred_element_type=jnp.float32) + b2_ref[...]
        ea_ref[...] = ea.astype(_BF)
        inv = 1.0 / (d + 1e-8)
        xs, ys, zs = x * inv, y * inv, z * inv
        one = jnp.ones_like(xs)
        cols = [one, xs, ys, zs, s3 * xs * ys, s3 * ys * zs,
                0.5 * (3.0 * zs * zs - 1.0), s3 * xs * zs,
                0.5 * s3 * (xs * xs - ys * ys),
                jnp.zeros((xs.shape[0], 7), jnp.float32)]
        sh_ref[...] = jnp.concatenate(cols, axis=-1).astype(_BF)

    din = W1.shape[0]
    in_specs = [pl.BlockSpec((_BE, 16), lambda i: (i, 0))]
    args = [vec_in]
    if ef is not None:
        in_specs.append(pl.BlockSpec((_BE, 4), lambda i: (i, 0)))
        args.append(ef)
    in_specs.append(pl.BlockSpec((1, 32), lambda i: (0, 0)))
    args.append(jnp.asarray(offs).reshape(1, 32))
    in_specs += [pl.BlockSpec((din, NS), lambda i: (0, 0)),
                 pl.BlockSpec((1, NS), lambda i: (0, 0)),
                 pl.BlockSpec((NS, NS), lambda i: (0, 0)),
                 pl.BlockSpec((1, NS), lambda i: (0, 0))]
    args += [W1.astype(_BF), b1.reshape(1, NS), W2.astype(_BF), b2.reshape(1, NS)]
    return pl.pallas_call(
        body,
        grid=(_NB_EDGE,),
        in_specs=in_specs,
        out_specs=[pl.BlockSpec((_BE, NS), lambda i: (i, 0)),
                   pl.BlockSpec((_BE, 16), lambda i: (i, 0))],
        out_shape=[jax.ShapeDtypeStruct((EP, NS), _BF),
                   jax.ShapeDtypeStruct((EP, 16), _BF)],
    )(*args)


def _tc_conv(ea, sh, ga, gb, fc1, fc1b, fc2, fc2b, srcW, shW, src_is_a, din, dout, gw):
    two_out = dout > D

    def body(ea_ref, sh_ref, ga_ref, gb_ref, w1_ref, b1_ref, w2_ref, b2_ref,
             sw_ref, hw_ref, *out_refs):
        ga_v = ga_ref[...].astype(_BF)
        gb_v = gb_ref[...].astype(_BF)
        xcat = jnp.concatenate([ea_ref[...], ga_v[:, :NS], gb_v[:, :NS]], axis=-1)
        h = jnp.maximum(jnp.dot(xcat, w1_ref[...],
                                preferred_element_type=jnp.float32)
                        + b1_ref[...], 0.0)
        w = jnp.dot(h.astype(_BF), w2_ref[...],
                    preferred_element_type=jnp.float32) + b2_ref[...]
        f = ga_v if src_is_a else gb_v
        msg = (w
               * jnp.dot(f, sw_ref[...], preferred_element_type=jnp.float32)
               * jnp.dot(sh_ref[...], hw_ref[...], preferred_element_type=jnp.float32))
        one = jnp.ones((_BE, 1), jnp.float32)
        if two_out:
            out_refs[0][...] = msg[:, :D]
            out_refs[1][...] = jnp.concatenate(
                [msg[:, D:], one, jnp.zeros((_BE, 2 * D - dout - 1), jnp.float32)],
                axis=-1)
        else:
            out_refs[0][...] = jnp.concatenate(
                [msg, one, jnp.zeros((_BE, D - dout - 1), jnp.float32)], axis=-1)

    shW16 = jnp.pad(shW, ((0, 7), (0, 0)))
    n_out = 2 if two_out else 1
    return pl.pallas_call(
        body,
        grid=(_NB_EDGE,),
        in_specs=[pl.BlockSpec((_BE, NS), lambda i: (i, 0)),
                  pl.BlockSpec((_BE, 16), lambda i: (i, 0)),
                  pl.BlockSpec((_BE, gw), lambda i: (i, 0)),
                  pl.BlockSpec((_BE, gw), lambda i: (i, 0)),
                  pl.BlockSpec((3 * NS, 3 * NS), lambda i: (0, 0)),
                  pl.BlockSpec((1, 3 * NS), lambda i: (0, 0)),
                  pl.BlockSpec((3 * NS, dout), lambda i: (0, 0)),
                  pl.BlockSpec((1, dout), lambda i: (0, 0)),
                  pl.BlockSpec((din, dout), lambda i: (0, 0)),
                  pl.BlockSpec((16, dout), lambda i: (0, 0))],
        out_specs=[pl.BlockSpec((_BE, D), lambda i: (i, 0))] * n_out,
        out_shape=[jax.ShapeDtypeStruct((EP, D), jnp.float32)] * n_out,
    )(ea, sh, ga, gb, fc1.astype(_BF), fc1b.reshape(1, -1),
      fc2.astype(_BF), fc2b.reshape(1, -1), srcW.astype(_BF), shW16.astype(_BF))


def _tc_combine0(na_old, intra, inter):
    def body(na_ref, ia_ref, ie_ref, out_ref):
        ia = ia_ref[0] + ia_ref[1]
        ie = ie_ref[0] + ie_ref[1]
        ca = jnp.maximum(ia[:, 112:113], 1.0)
        ce = jnp.maximum(ie[:, 112:113], 1.0)
        old = jnp.concatenate(
            [na_ref[...], jnp.zeros((_BN, 112 - NS), jnp.float32)], axis=-1)
        out_ref[...] = old + ia[:, :112] / ca + ie[:, :112] / ce

    return pl.pallas_call(
        body,
        grid=(_NB_NODE,),
        in_specs=[pl.BlockSpec((_BN, NS), lambda i: (i, 0)),
                  pl.BlockSpec((NSC, _BN, D), lambda i: (0, i, 0)),
                  pl.BlockSpec((NSC, _BN, D), lambda i: (0, i, 0))],
        out_specs=pl.BlockSpec((_BN, 112), lambda i: (i, 0)),
        out_shape=jax.ShapeDtypeStruct((NL, 112), jnp.float32),
    )(na_old, intra, inter)


def _tc_combine1(na_old, intra_a, intra_b, inter_a, inter_b):
    def body(na_ref, iaa_ref, iab_ref, iea_ref, ieb_ref, out_ref):
        iaa = iaa_ref[0] + iaa_ref[1]
        iab = iab_ref[0] + iab_ref[1]
        iea = iea_ref[0] + iea_ref[1]
        ieb = ieb_ref[0] + ieb_ref[1]
        ca = jnp.maximum(iab[:, 32:33], 1.0)
        ce = jnp.maximum(ieb[:, 32:33], 1.0)
        ia = jnp.concatenate([iaa, iab[:, :32]], axis=-1)
        ie = jnp.concatenate([iea, ieb[:, :32]], axis=-1)
        old = jnp.concatenate(
            [na_ref[...], jnp.zeros((_BN, 160 - 112), jnp.float32)], axis=-1)
        out_ref[...] = old + ia / ca + ie / ce

    return pl.pallas_call(
        body,
        grid=(_NB_NODE,),
        in_specs=[pl.BlockSpec((_BN, 112), lambda i: (i, 0)),
                  pl.BlockSpec((NSC, _BN, D), lambda i: (0, i, 0)),
                  pl.BlockSpec((NSC, _BN, D), lambda i: (0, i, 0)),
                  pl.BlockSpec((NSC, _BN, D), lambda i: (0, i, 0)),
                  pl.BlockSpec((NSC, _BN, D), lambda i: (0, i, 0))],
        out_specs=pl.BlockSpec((_BN, 160), lambda i: (i, 0)),
        out_shape=jax.ShapeDtypeStruct((NL, 160), jnp.float32),
    )(na_old, intra_a, intra_b, inter_a, inter_b)


def _pad_idx(idx, fill):
    idx = jnp.concatenate(
        [idx.astype(jnp.int32), jnp.full((EP - E,), fill, jnp.int32)])
    return idx.reshape(EP // CH, CH)


def kernel(lig_x, rec_x, lig_pos, rec_pos, lig_edge_index, lig_edge_feats,
           rec_edge_index, cross_edge_index, emb_lig, emb_rec,
           lig_eW1, lig_eb1, lig_eW2, lig_eb2,
           rec_eW1, rec_eb1, rec_eW2, rec_eb2,
           cr_eW1, cr_eb1, cr_eW2, cr_eb2,
           l0_fc1, l0_fc1b, l0_fc2, l0_fc2b, l0_srcW, l0_shW,
           l1_fc1, l1_fc1b, l1_fc2, l1_fc2b, l1_srcW, l1_shW):
    ls, ld = lig_edge_index[0], lig_edge_index[1]
    rs, rd = rec_edge_index[0], rec_edge_index[1]
    cl, cr = cross_edge_index[0], cross_edge_index[1]

    g_ls, g_ld = _pad_idx(ls, 0), _pad_idx(ld, 0)
    g_rs, g_rd = _pad_idx(rs, 0), _pad_idx(rd, 0)
    g_cl, g_cr = _pad_idx(cl, 0), _pad_idx(cr, 0)
    s_ld = _pad_idx(ld, NP - 1)
    s_rd = _pad_idx(rd, NP - 1)
    s_cl = _pad_idx(cl, NP - 1)
    s_cr = _pad_idx(cr, NP - 1)

    lig_pos128 = jnp.pad(lig_pos, ((0, 0), (0, 13)))
    rec_pos128 = jnp.pad(rec_pos, ((0, 0), (0, 13)))
    ef_pad = jnp.pad(lig_edge_feats, ((0, EP - E), (0, 0)))
    z128 = jnp.zeros((NP, D), jnp.float32)

    lig_vec = _sc_edge_vec(lig_pos128, lig_pos128, g_ls, g_ld)
    rec_vec = _sc_edge_vec(rec_pos128, rec_pos128, g_rs, g_rd)
    cr_vec = _sc_edge_vec(lig_pos128, rec_pos128, g_cl, g_cr)

    lig_ea, lig_sh = _tc_geometry(lig_vec, ef_pad, lig_eW1, lig_eb1,
                                  lig_eW2, lig_eb2, 0.0, 5.0)
    rec_ea, rec_sh = _tc_geometry(rec_vec, None, rec_eW1, rec_eb1,
                                  rec_eW2, rec_eb2, 0.0, 30.0)
    cr_ea, cr_sh = _tc_geometry(cr_vec, None, cr_eW1, cr_eb1,
                                cr_eW2, cr_eb2, 0.0, 250.0)

    lig_na = _tc_atom_enc(lig_x.astype(jnp.int32), emb_lig)
    rec_na = _tc_atom_enc(rec_x.astype(jnp.int32), emb_rec)

    layers = [(l0_fc1, l0_fc1b, l0_fc2, l0_fc2b, l0_srcW, l0_shW),
              (l1_fc1, l1_fc1b, l1_fc2, l1_fc2b, l1_srcW, l1_shW)]
    for l in range(2):
        fc1, fc1b, fc2, fc2b, srcW, shW = layers[l]
        din, dout = DIMS[l], DIMS[l + 1]
        gw = din
        ga_ls = _sc_gather(lig_na, g_ls, gw)
        ga_ld = _sc_gather(lig_na, g_ld, gw)
        ga_cl = _sc_gather(lig_na, g_cl, gw)
        ga_cr = _sc_gather(rec_na, g_cr, gw)

        if l == 0:
            m_li, = _tc_conv(lig_ea, lig_sh, ga_ls, ga_ld, fc1[0], fc1b[0],
                             fc2[0], fc2b[0], srcW[0], shW[0], True, din, dout, gw)
            m_lx, = _tc_conv(cr_ea, cr_sh, ga_cl, ga_cr, fc1[3], fc1b[3],
                             fc2[3], fc2b[3], srcW[3], shW[3], False, din, dout, gw)
            lig_intra = _sc_scatter_add(m_li, s_ld, z128)
            lig_inter = _sc_scatter_add(m_lx, s_cl, z128)

            ga_rs = _sc_gather(rec_na, g_rs, gw)
            ga_rd = _sc_gather(rec_na, g_rd, gw)
            m_ri, = _tc_conv(rec_ea, rec_sh, ga_rs, ga_rd, fc1[1], fc1b[1],
                             fc2[1], fc2b[1], srcW[1], shW[1], True, din, dout, gw)
            m_rx, = _tc_conv(cr_ea, cr_sh, ga_cl, ga_cr, fc1[2], fc1b[2],
                             fc2[2], fc2b[2], srcW[2], shW[2], True, din, dout, gw)
            rec_intra = _sc_scatter_add(m_ri, s_rd, z128)
            rec_inter = _sc_scatter_add(m_rx, s_cr, z128)
            rec_na = _tc_combine0(rec_na, rec_intra, rec_inter)
            lig_na = _tc_combine0(lig_na, lig_intra, lig_inter)
        else:
            mA_li, mB_li = _tc_conv(lig_ea, lig_sh, ga_ls, ga_ld, fc1[0], fc1b[0],
                                    fc2[0], fc2b[0], srcW[0], shW[0], True, din, dout, gw)
            mA_lx, mB_lx = _tc_conv(cr_ea, cr_sh, ga_cl, ga_cr, fc1[3], fc1b[3],
                                    fc2[3], fc2b[3], srcW[3], shW[3], False, din, dout, gw)
            ia_a = _sc_scatter_add(mA_li, s_ld, z128)
            ia_b = _sc_scatter_add(mB_li, s_ld, z128)
            ie_a = _sc_scatter_add(mA_lx, s_cl, z128)
            ie_b = _sc_scatter_add(mB_lx, s_cl, z128)
            lig_na = _tc_combine1(lig_na, ia_a, ia_b, ie_a, ie_b)

    return lig_na


# speedup vs baseline: 1.0731x; 1.0731x over previous
import functools

import numpy as np
import jax
import jax.numpy as jnp
from jax import lax
from jax.experimental import pallas as pl
from jax.experimental.pallas import tpu as pltpu
from jax.experimental.pallas import tpu_sc as plsc

NS = 64
NCAT = 8
NL = 10000
NR = 10000
E = 160000
DIMS = [64, 112, 160]

NSC = 2
NSUB = 16
NWORK = NSC * NSUB
CH = 128
EP = 163840
NCH_W = EP // CH // NWORK
W_ROWS = EP // NWORK
NP = 10240
ACC_ROWS = NP // NSUB
D = 128

_SC_MESH = dict(core_axis_name="c", subcore_axis_name="s",
                num_cores=NSC, num_subcores=NSUB)


def _sc_gather(table, idx2d, gw=D):
    mesh = plsc.VectorSubcoreMesh(**_SC_MESH)

    @functools.partial(
        pl.kernel,
        out_type=jax.ShapeDtypeStruct((EP, D), jnp.float32),
        mesh=mesh,
        scratch_types=[
            pltpu.VMEM((NCH_W, CH), jnp.int32),
            pltpu.VMEM((CH, D), jnp.float32),
            pltpu.VMEM((CH, D), jnp.float32),
            pltpu.VMEM((CH, D), jnp.float32),
            pltpu.VMEM((CH, D), jnp.float32),
            pltpu.SemaphoreType.DMA,
            pltpu.SemaphoreType.DMA,
            pltpu.SemaphoreType.DMA,
            pltpu.SemaphoreType.DMA,
        ],
    )
    def k(table_h, idx_h, out_h, idx_v, b0, b1, b2, b3, s0, s1, s2, s3):
        cid = lax.axis_index("c")
        sid = lax.axis_index("s")
        wid = sid * NSC + cid
        pltpu.sync_copy(idx_h.at[pl.ds(wid * NCH_W, NCH_W)], idx_v)
        base = wid * W_ROWS
        bufs = (b0, b1, b2, b3)
        sems = (s0, s1, s2, s3)
        for t in range(3):
            pltpu.async_copy(table_h.at[idx_v.at[t]], bufs[t], sems[t])

        def body(i, carry):
            for t in range(4):
                j = 4 * i + t
                bn = (t + 3) % 4

                @pl.when(j + 3 < NCH_W)
                def _():
                    pltpu.async_copy(table_h.at[idx_v.at[j + 3]], bufs[bn], sems[bn])

                pltpu.make_async_copy(table_h.at[idx_v.at[j]], bufs[t], sems[t]).wait()
                pltpu.sync_copy(bufs[t], out_h.at[pl.ds(base + j * CH, CH)])
            return carry

        lax.fori_loop(0, NCH_W // 4, body, 0)

    return k(table, idx2d)


def _sc_edge_vec(tab_s, tab_d, idxs2d, idxd2d):
    mesh = plsc.VectorSubcoreMesh(**_SC_MESH)

    @functools.partial(
        pl.kernel,
        out_type=jax.ShapeDtypeStruct((EP, 16), jnp.float32),
        mesh=mesh,
        compiler_params=pltpu.CompilerParams(use_tc_tiling_on_sc=False),
        scratch_types=[
            pltpu.VMEM((NCH_W, CH), jnp.int32),
            pltpu.VMEM((NCH_W, CH), jnp.int32),
            pltpu.VMEM((CH, 16), jnp.float32),
            pltpu.VMEM((CH, 16), jnp.float32),
            pltpu.VMEM((CH, 16), jnp.float32),
            pltpu.VMEM((CH, 16), jnp.float32),
            pltpu.VMEM((CH, 16), jnp.float32),
            pltpu.VMEM((CH, 16), jnp.float32),
            pltpu.SemaphoreType.DMA,
            pltpu.SemaphoreType.DMA,
            pltpu.SemaphoreType.DMA,
            pltpu.SemaphoreType.DMA,
        ],
    )
    def k(tabs_h, tabd_h, idxs_h, idxd_h, out_h, idxs_v, idxd_v,
          bs0, bd0, bs1, bd1, ov0, ov1, ss0, sd0, ss1, sd1):
        cid = lax.axis_index("c")
        sid = lax.axis_index("s")
        wid = sid * NSC + cid
        pltpu.sync_copy(idxs_h.at[pl.ds(wid * NCH_W, NCH_W)], idxs_v)
        pltpu.sync_copy(idxd_h.at[pl.ds(wid * NCH_W, NCH_W)], idxd_v)
        base = wid * W_ROWS
        pltpu.async_copy(tabs_h.at[idxs_v.at[0]], bs0, ss0)
        pltpu.async_copy(tabd_h.at[idxd_v.at[0]], bd0, sd0)

        def diff(bs, bd, ov):
            def row(r, carry):
                for u in range(4):
                    ov[4 * r + u, 0:16] = bd[4 * r + u, 0:16] - bs[4 * r + u, 0:16]
                return carry
            lax.fori_loop(0, CH // 4, row, 0)

        def body(i, carry):
            j0 = 2 * i
            pltpu.async_copy(tabs_h.at[idxs_v.at[j0 + 1]], bs1, ss1)
            pltpu.async_copy(tabd_h.at[idxd_v.at[j0 + 1]], bd1, sd1)
            pltpu.make_async_copy(tabs_h.at[idxs_v.at[j0]], bs0, ss0).wait()
            pltpu.make_async_copy(tabd_h.at[idxd_v.at[j0]], bd0, sd0).wait()
            diff(bs0, bd0, ov0)
            pltpu.sync_copy(ov0, out_h.at[pl.ds(base + j0 * CH, CH)])

            @pl.when(j0 + 2 < NCH_W)
            def _():
                pltpu.async_copy(tabs_h.at[idxs_v.at[j0 + 2]], bs0, ss0)
                pltpu.async_copy(tabd_h.at[idxd_v.at[j0 + 2]], bd0, sd0)

            pltpu.make_async_copy(tabs_h.at[idxs_v.at[j0 + 1]], bs1, ss1).wait()
            pltpu.make_async_copy(tabd_h.at[idxd_v.at[j0 + 1]], bd1, sd1).wait()
            diff(bs1, bd1, ov1)
            pltpu.sync_copy(ov1, out_h.at[pl.ds(base + (j0 + 1) * CH, CH)])
            return carry

        lax.fori_loop(0, NCH_W // 2, body, 0)

    return k(tab_s, tab_d, idxs2d, idxd2d)


def _sc_scatter_add(msg, idx2d, zeros_np):
    mesh = plsc.VectorSubcoreMesh(**_SC_MESH)

    @functools.partial(
        pl.kernel,
        out_type=jax.ShapeDtypeStruct((NSC, NP, D), jnp.float32),
        mesh=mesh,
        scratch_types=[
            pltpu.VMEM((NCH_W, CH), jnp.int32),
            pltpu.VMEM((CH, D), jnp.float32),
            pltpu.VMEM((CH, D), jnp.float32),
            pltpu.VMEM_SHARED((NP, D), jnp.float32),
            pltpu.SemaphoreType.DMA,
            pltpu.SemaphoreType.DMA,
        ],
    )
    def k(msg_h, idx_h, zero_h, out_h, idx_v, b0, b1, acc, s0, s1):
        cid = lax.axis_index("c")
        sid = lax.axis_index("s")
        wid = cid * NSUB + sid
        pltpu.sync_copy(idx_h.at[pl.ds(wid * NCH_W, NCH_W)], idx_v)
        pltpu.sync_copy(zero_h.at[pl.ds(sid * ACC_ROWS, ACC_ROWS)],
                        acc.at[pl.ds(sid * ACC_ROWS, ACC_ROWS)])
        plsc.subcore_barrier()
        base = wid * W_ROWS
        bufs = (b0, b1)
        sems = (s0, s1)
        pltpu.async_copy(msg_h.at[pl.ds(base, CH)], b0, s0)

        def body(i, carry):
            for t in range(2):
                j = 2 * i + t
                bn = (t + 1) % 2

                @pl.when(j + 1 < NCH_W)
                def _():
                    pltpu.async_copy(msg_h.at[pl.ds(base + (j + 1) * CH, CH)],
                                     bufs[bn], sems[bn])

                pltpu.make_async_copy(msg_h.at[pl.ds(base, CH)], bufs[t], sems[t]).wait()
                pltpu.sync_copy(bufs[t], acc.at[idx_v.at[j]], add=True)
            return carry

        lax.fori_loop(0, NCH_W // 2, body, 0)
        plsc.subcore_barrier()
        pltpu.sync_copy(acc.at[pl.ds(sid * ACC_ROWS, ACC_ROWS)],
                        out_h.at[cid, pl.ds(sid * ACC_ROWS, ACC_ROWS)])

    return k(msg, idx2d, zeros_np)


_NB_NODE = 5
_BN = NL // _NB_NODE
_BE = 2048
_NB_EDGE = EP // _BE

_BF = jnp.bfloat16


def _tc_atom_enc(x, emb):
    def body(x_ref, emb_ref, out_ref):
        xv = x_ref[...]
        acc = jnp.zeros((_BN, NS), jnp.float32)
        ids = lax.broadcasted_iota(jnp.int32, (_BN, 16), 1)
        for i in range(NCAT):
            oh = (xv[:, i:i + 1] == ids).astype(_BF)
            acc = acc + jnp.dot(oh, emb_ref[i], preferred_element_type=jnp.float32)
        out_ref[...] = jnp.concatenate(
            [acc, jnp.zeros((_BN, D - NS), jnp.float32)], axis=-1)

    return pl.pallas_call(
        body,
        grid=(_NB_NODE,),
        in_specs=[pl.BlockSpec((_BN, NCAT), lambda i: (i, 0)),
                  pl.BlockSpec((NCAT, 16, NS), lambda i: (0, 0, 0))],
        out_specs=pl.BlockSpec((_BN, D), lambda i: (i, 0)),
        out_shape=jax.ShapeDtypeStruct((NL, D), jnp.float32),
    )(x, emb.astype(_BF))


def _tc_geometry(vec_in, ef, W1, b1, W2, b2, start, stop):
    offs = np.linspace(start, stop, 32).astype(np.float32)
    coeff = float(-0.5 / (offs[1] - offs[0]) ** 2)
    s3 = float(np.sqrt(3.0))

    def body(*refs):
        if ef is not None:
            vec_ref, ef_ref, offs_ref, w1_ref, b1_ref, w2_ref, b2_ref, ea_ref, sh_ref = refs
        else:
            vec_ref, offs_ref, w1_ref, b1_ref, w2_ref, b2_ref, ea_ref, sh_ref = refs
        vec = vec_ref[...]
        x = vec[:, 0:1]
        y = vec[:, 1:2]
        z = vec[:, 2:3]
        d2 = x * x + y * y + z * z
        d = jnp.sqrt(d2 + 1e-12)
        g = jnp.exp(coeff * (d - offs_ref[...]) ** 2)
        if ef is not None:
            g = jnp.concatenate([ef_ref[...], g], axis=-1)
        h = jnp.maximum(jnp.dot(g.astype(_BF), w1_ref[...],
                                preferred_element_type=jnp.float32)
                        + b1_ref[...], 0.0)
        ea = jnp.dot(h.astype(_BF), w2_ref[...],
                     preferred_element_type=jnp.float32) + b2_ref[...]
        ea_ref[...] = ea.astype(_BF)
        inv = 1.0 / (d + 1e-8)
        xs, ys, zs = x * inv, y * inv, z * inv
        one = jnp.ones_like(xs)
        cols = [one, xs, ys, zs, s3 * xs * ys, s3 * ys * zs,
                0.5 * (3.0 * zs * zs - 1.0), s3 * xs * zs,
                0.5 * s3 * (xs * xs - ys * ys),
                jnp.zeros((xs.shape[0], 7), jnp.float32)]
        sh_ref[...] = jnp.concatenate(cols, axis=-1).astype(_BF)

    din = W1.shape[0]
    in_specs = [pl.BlockSpec((_BE, 16), lambda i: (i, 0))]
    args = [vec_in]
    if ef is not None:
        in_specs.append(pl.BlockSpec((_BE, 4), lambda i: (i, 0)))
        args.append(ef)
    in_specs.append(pl.BlockSpec((1, 32), lambda i: (0, 0)))
    args.append(jnp.asarray(offs).reshape(1, 32))
    in_specs += [pl.BlockSpec((din, NS), lambda i: (0, 0)),
                 pl.BlockSpec((1, NS), lambda i: (0, 0)),
                 pl.BlockSpec((NS, NS), lambda i: (0, 0)),
                 pl.BlockSpec((1, NS), lambda i: (0, 0))]
    args += [W1.astype(_BF), b1.reshape(1, NS), W2.astype(_BF), b2.reshape(1, NS)]
    return pl.pallas_call(
        body,
        grid=(_NB_EDGE,),
        in_specs=in_specs,
        out_specs=[pl.BlockSpec((_BE, NS), lambda i: (i, 0)),
                   pl.BlockSpec((_BE, 16), lambda i: (i, 0))],
        out_shape=[jax.ShapeDtypeStruct((EP, NS), _BF),
                   jax.ShapeDtypeStruct((EP, 16), _BF)],
    )(*args)


def _tc_conv(ea, sh, ga, gb, fc1, fc1b, fc2, fc2b, srcW, shW, src_is_a, din, dout, gw):
    two_out = dout > D

    def body(ea_ref, sh_ref, ga_ref, gb_ref, w1_ref, b1_ref, w2_ref, b2_ref,
             sw_ref, hw_ref, *out_refs):
        ga_v = ga_ref[...].astype(_BF)
        gb_v = gb_ref[...].astype(_BF)
        xcat = jnp.concatenate([ea_ref[...], ga_v[:, :NS], gb_v[:, :NS]], axis=-1)
        h = jnp.maximum(jnp.dot(xcat, w1_ref[...],
                                preferred_element_type=jnp.float32)
                        + b1_ref[...], 0.0)
        w = jnp.dot(h.astype(_BF), w2_ref[...],
                    preferred_element_type=jnp.float32) + b2_ref[...]
        f = (ga_v if src_is_a else gb_v)[:, :din]
        msg = (w
               * jnp.dot(f, sw_ref[...], preferred_element_type=jnp.float32)
               * jnp.dot(sh_ref[...], hw_ref[...], preferred_element_type=jnp.float32))
        one = jnp.ones((_BE, 1), jnp.float32)
        if two_out:
            out_refs[0][...] = msg[:, :D]
            out_refs[1][...] = jnp.concatenate(
                [msg[:, D:], one, jnp.zeros((_BE, 2 * D - dout - 1), jnp.float32)],
                axis=-1)
        else:
            out_refs[0][...] = jnp.concatenate(
                [msg, one, jnp.zeros((_BE, D - dout - 1), jnp.float32)], axis=-1)

    shW16 = jnp.pad(shW, ((0, 7), (0, 0)))
    n_out = 2 if two_out else 1
    return pl.pallas_call(
        body,
        grid=(_NB_EDGE,),
        in_specs=[pl.BlockSpec((_BE, NS), lambda i: (i, 0)),
                  pl.BlockSpec((_BE, 16), lambda i: (i, 0)),
                  pl.BlockSpec((_BE, gw), lambda i: (i, 0)),
                  pl.BlockSpec((_BE, gw), lambda i: (i, 0)),
                  pl.BlockSpec((3 * NS, 3 * NS), lambda i: (0, 0)),
                  pl.BlockSpec((1, 3 * NS), lambda i: (0, 0)),
                  pl.BlockSpec((3 * NS, dout), lambda i: (0, 0)),
                  pl.BlockSpec((1, dout), lambda i: (0, 0)),
                  pl.BlockSpec((din, dout), lambda i: (0, 0)),
                  pl.BlockSpec((16, dout), lambda i: (0, 0))],
        out_specs=[pl.BlockSpec((_BE, D), lambda i: (i, 0))] * n_out,
        out_shape=[jax.ShapeDtypeStruct((EP, D), jnp.float32)] * n_out,
    )(ea, sh, ga, gb, fc1.astype(_BF), fc1b.reshape(1, -1),
      fc2.astype(_BF), fc2b.reshape(1, -1), srcW.astype(_BF), shW16.astype(_BF))


def _tc_combine0(na_old, intra, inter):
    def body(na_ref, ia_ref, ie_ref, out_ref):
        ia = ia_ref[0] + ia_ref[1]
        ie = ie_ref[0] + ie_ref[1]
        ca = jnp.maximum(ia[:, 112:113], 1.0)
        ce = jnp.maximum(ie[:, 112:113], 1.0)
        old = jnp.concatenate(
            [na_ref[...][:, :NS], jnp.zeros((_BN, 112 - NS), jnp.float32)],
            axis=-1)
        upd = old + ia[:, :112] / ca + ie[:, :112] / ce
        out_ref[...] = jnp.concatenate(
            [upd, jnp.zeros((_BN, 16), jnp.float32)], axis=-1)

    return pl.pallas_call(
        body,
        grid=(_NB_NODE,),
        in_specs=[pl.BlockSpec((_BN, D), lambda i: (i, 0)),
                  pl.BlockSpec((NSC, _BN, D), lambda i: (0, i, 0)),
                  pl.BlockSpec((NSC, _BN, D), lambda i: (0, i, 0))],
        out_specs=pl.BlockSpec((_BN, D), lambda i: (i, 0)),
        out_shape=jax.ShapeDtypeStruct((NL, D), jnp.float32),
    )(na_old, intra, inter)


def _tc_combine1(na_old, intra_a, intra_b, inter_a, inter_b):
    def body(na_ref, iaa_ref, iab_ref, iea_ref, ieb_ref, out_ref):
        iaa = iaa_ref[0] + iaa_ref[1]
        iab = iab_ref[0] + iab_ref[1]
        iea = iea_ref[0] + iea_ref[1]
        ieb = ieb_ref[0] + ieb_ref[1]
        ca = jnp.maximum(iab[:, 32:33], 1.0)
        ce = jnp.maximum(ieb[:, 32:33], 1.0)
        ia = jnp.concatenate([iaa, iab[:, :32]], axis=-1)
        ie = jnp.concatenate([iea, ieb[:, :32]], axis=-1)
        old = jnp.concatenate(
            [na_ref[...][:, :112], jnp.zeros((_BN, 160 - 112), jnp.float32)],
            axis=-1)
        out_ref[...] = old + ia / ca + ie / ce

    return pl.pallas_call(
        body,
        grid=(_NB_NODE,),
        in_specs=[pl.BlockSpec((_BN, D), lambda i: (i, 0)),
                  pl.BlockSpec((NSC, _BN, D), lambda i: (0, i, 0)),
                  pl.BlockSpec((NSC, _BN, D), lambda i: (0, i, 0)),
                  pl.BlockSpec((NSC, _BN, D), lambda i: (0, i, 0)),
                  pl.BlockSpec((NSC, _BN, D), lambda i: (0, i, 0))],
        out_specs=pl.BlockSpec((_BN, 160), lambda i: (i, 0)),
        out_shape=jax.ShapeDtypeStruct((NL, 160), jnp.float32),
    )(na_old, intra_a, intra_b, inter_a, inter_b)


def _pad_idx(idx, fill):
    idx = jnp.concatenate(
        [idx.astype(jnp.int32), jnp.full((EP - E,), fill, jnp.int32)])
    return idx.reshape(EP // CH, CH)


def kernel(lig_x, rec_x, lig_pos, rec_pos, lig_edge_index, lig_edge_feats,
           rec_edge_index, cross_edge_index, emb_lig, emb_rec,
           lig_eW1, lig_eb1, lig_eW2, lig_eb2,
           rec_eW1, rec_eb1, rec_eW2, rec_eb2,
           cr_eW1, cr_eb1, cr_eW2, cr_eb2,
           l0_fc1, l0_fc1b, l0_fc2, l0_fc2b, l0_srcW, l0_shW,
           l1_fc1, l1_fc1b, l1_fc2, l1_fc2b, l1_srcW, l1_shW):
    ls, ld = lig_edge_index[0], lig_edge_index[1]
    rs, rd = rec_edge_index[0], rec_edge_index[1]
    cl, cr = cross_edge_index[0], cross_edge_index[1]

    g_ls, g_ld = _pad_idx(ls, 0), _pad_idx(ld, 0)
    g_rs, g_rd = _pad_idx(rs, 0), _pad_idx(rd, 0)
    g_cl, g_cr = _pad_idx(cl, 0), _pad_idx(cr, 0)
    s_ld = _pad_idx(ld, NP - 1)
    s_rd = _pad_idx(rd, NP - 1)
    s_cl = _pad_idx(cl, NP - 1)
    s_cr = _pad_idx(cr, NP - 1)

    lig_pos128 = jnp.pad(lig_pos, ((0, 0), (0, 13)))
    rec_pos128 = jnp.pad(rec_pos, ((0, 0), (0, 13)))
    ef_pad = jnp.pad(lig_edge_feats, ((0, EP - E), (0, 0)))
    z128 = jnp.zeros((NP, D), jnp.float32)

    lig_vec = _sc_edge_vec(lig_pos128, lig_pos128, g_ls, g_ld)
    rec_vec = _sc_edge_vec(rec_pos128, rec_pos128, g_rs, g_rd)
    cr_vec = _sc_edge_vec(lig_pos128, rec_pos128, g_cl, g_cr)

    lig_ea, lig_sh = _tc_geometry(lig_vec, ef_pad, lig_eW1, lig_eb1,
                                  lig_eW2, lig_eb2, 0.0, 5.0)
    rec_ea, rec_sh = _tc_geometry(rec_vec, None, rec_eW1, rec_eb1,
                                  rec_eW2, rec_eb2, 0.0, 30.0)
    cr_ea, cr_sh = _tc_geometry(cr_vec, None, cr_eW1, cr_eb1,
                                cr_eW2, cr_eb2, 0.0, 250.0)

    lig_na = _tc_atom_enc(lig_x.astype(jnp.int32), emb_lig)
    rec_na = _tc_atom_enc(rec_x.astype(jnp.int32), emb_rec)

    layers = [(l0_fc1, l0_fc1b, l0_fc2, l0_fc2b, l0_srcW, l0_shW),
              (l1_fc1, l1_fc1b, l1_fc2, l1_fc2b, l1_srcW, l1_shW)]
    for l in range(2):
        fc1, fc1b, fc2, fc2b, srcW, shW = layers[l]
        din, dout = DIMS[l], DIMS[l + 1]
        gw = D
        ga_ls = _sc_gather(lig_na, g_ls, gw)
        ga_ld = _sc_gather(lig_na, g_ld, gw)
        ga_cl = _sc_gather(lig_na, g_cl, gw)
        ga_cr = _sc_gather(rec_na, g_cr, gw)

        if l == 0:
            m_li, = _tc_conv(lig_ea, lig_sh, ga_ls, ga_ld, fc1[0], fc1b[0],
                             fc2[0], fc2b[0], srcW[0], shW[0], True, din, dout, gw)
            m_lx, = _tc_conv(cr_ea, cr_sh, ga_cl, ga_cr, fc1[3], fc1b[3],
                             fc2[3], fc2b[3], srcW[3], shW[3], False, din, dout, gw)
            lig_intra = _sc_scatter_add(m_li, s_ld, z128)
            lig_inter = _sc_scatter_add(m_lx, s_cl, z128)

            ga_rs = _sc_gather(rec_na, g_rs, gw)
            ga_rd = _sc_gather(rec_na, g_rd, gw)
            m_ri, = _tc_conv(rec_ea, rec_sh, ga_rs, ga_rd, fc1[1], fc1b[1],
                             fc2[1], fc2b[1], srcW[1], shW[1], True, din, dout, gw)
            m_rx, = _tc_conv(cr_ea, cr_sh, ga_cl, ga_cr, fc1[2], fc1b[2],
                             fc2[2], fc2b[2], srcW[2], shW[2], True, din, dout, gw)
            rec_intra = _sc_scatter_add(m_ri, s_rd, z128)
            rec_inter = _sc_scatter_add(m_rx, s_cr, z128)
            rec_na = _tc_combine0(rec_na, rec_intra, rec_inter)
            lig_na = _tc_combine0(lig_na, lig_intra, lig_inter)
        else:
            mA_li, mB_li = _tc_conv(lig_ea, lig_sh, ga_ls, ga_ld, fc1[0], fc1b[0],
                                    fc2[0], fc2b[0], srcW[0], shW[0], True, din, dout, gw)
            mA_lx, mB_lx = _tc_conv(cr_ea, cr_sh, ga_cl, ga_cr, fc1[3], fc1b[3],
                                    fc2[3], fc2b[3], srcW[3], shW[3], False, din, dout, gw)
            ia_a = _sc_scatter_add(mA_li, s_ld, z128)
            ia_b = _sc_scatter_add(mB_li, s_ld, z128)
            ie_a = _sc_scatter_add(mA_lx, s_cl, z128)
            ie_b = _sc_scatter_add(mB_lx, s_cl, z128)
            lig_na = _tc_combine1(lig_na, ia_a, ia_b, ie_a, ie_b)

    return lig_na
